# Initial kernel scaffold; baseline (speedup 1.0000x reference)
import jax
import jax.numpy as jnp
from jax.experimental import pallas as pl


def kernel(flat, cu_seqlens):
    raise NotImplementedError("write your pallas kernel here")


# speedup vs baseline: 1.7888x; 1.7888x over previous
import functools

import jax
import jax.numpy as jnp
from jax import lax
from jax.experimental import pallas as pl
from jax.experimental.pallas import tpu as pltpu
from jax.experimental.pallas import tpu_sc as plsc

B = 16
MAX_LEN = 4096
D = 512
TOTAL = 32768

NC = 2
NS = 16
NW = NC * NS
HALF = (B * MAX_LEN) // NW
CH = 64
LADDER = (32, 16, 8, 4, 2, 1)


def _body(flat_hbm, desc_hbm, out_hbm, descv, buf, zbuf):
    c = lax.axis_index("c")
    s = lax.axis_index("s")
    wid = s * NC + c
    out_base = wid * HALF

    pltpu.sync_copy(desc_hbm.at[wid], descv)
    v = descv[...]
    lanes = lax.iota(jnp.int32, 16)
    zero16 = jnp.zeros((16,), jnp.int32)
    src0 = jnp.sum(jnp.where(lanes == 0, v, zero16))
    nvalid = jnp.sum(jnp.where(lanes == 1, v, zero16))

    zrow = jnp.zeros((16,), jnp.float32)

    def zr(i, carry):
        for k in range(D // 16):
            zbuf[i, pl.ds(k * 16, 16)] = zrow
        return carry

    lax.fori_loop(0, CH, zr, 0)

    n_full = nvalid // CH
    rem = nvalid - n_full * CH

    def cp(j, carry):
        pltpu.sync_copy(flat_hbm.at[pl.ds(src0 + j * CH, CH)], buf)
        pltpu.sync_copy(buf, out_hbm.at[pl.ds(out_base + j * CH, CH)])
        return carry

    lax.fori_loop(0, n_full, cp, 0)

    src_c = src0 + n_full * CH
    dst_c = out_base + n_full * CH
    for kb in LADDER:
        off = (rem // (2 * kb)) * (2 * kb)

        @pl.when((rem & kb) != 0)
        def _(off=off, kb=kb):
            pltpu.sync_copy(flat_hbm.at[pl.ds(src_c + off, kb)],
                            buf.at[pl.ds(0, kb)])
            pltpu.sync_copy(buf.at[pl.ds(0, kb)],
                            out_hbm.at[pl.ds(dst_c + off, kb)])

    pre = (CH - rem) % CH
    zdst = out_base + nvalid
    for kb in LADDER:
        off = (pre // (2 * kb)) * (2 * kb)

        @pl.when((pre & kb) != 0)
        def _(off=off, kb=kb):
            pltpu.sync_copy(zbuf.at[pl.ds(0, kb)],
                            out_hbm.at[pl.ds(zdst + off, kb)])

    zc0 = (nvalid + pre) // CH

    def zcp(j, carry):
        pltpu.sync_copy(zbuf, out_hbm.at[pl.ds(out_base + j * CH, CH)])
        return carry

    lax.fori_loop(zc0, HALF // CH, zcp, 0)


@jax.jit
def kernel(flat, cu_seqlens):
    cu = cu_seqlens.astype(jnp.int32)
    w = jnp.arange(NW, dtype=jnp.int32)
    b = w // 2
    p0 = (w % 2) * HALF
    src0 = cu[b] + p0
    nvalid = jnp.clip(cu[b + 1] - cu[b] - p0, 0, HALF)
    desc = jnp.zeros((NW, 16), jnp.int32)
    desc = desc.at[:, 0].set(src0).at[:, 1].set(nvalid)

    run = pl.kernel(
        _body,
        out_type=jax.ShapeDtypeStruct((B * MAX_LEN, D), jnp.float32),
        mesh=plsc.VectorSubcoreMesh(core_axis_name="c", subcore_axis_name="s"),
        scratch_types=[
            pltpu.VMEM((16,), jnp.int32),
            pltpu.VMEM((CH, D), jnp.float32),
            pltpu.VMEM((CH, D), jnp.float32),
        ],
        compiler_params=pltpu.CompilerParams(
            use_tc_tiling_on_sc=False, needs_layout_passes=False),
    )
    dense = run(flat, desc)
    return dense.reshape(B, MAX_LEN, D)


# speedup vs baseline: 1.9024x; 1.0635x over previous
import jax
import jax.numpy as jnp
from jax import lax
from jax.experimental import pallas as pl
from jax.experimental.pallas import tpu as pltpu
from jax.experimental.pallas import tpu_sc as plsc

B = 16
MAX_LEN = 4096
D = 512
TOTAL = 32768

NC = 2
NS = 16
NW = NC * NS
HALF = (B * MAX_LEN) // NW
CH = 64
LADDER = (32, 16, 8, 4, 2, 1)


def _body(flat_hbm, desc_hbm, out_hbm, descv, buf_a, buf_b, zbuf,
          sem_ra, sem_rb, sem_wa, sem_wb, sem_z, sem_l):
    c = lax.axis_index("c")
    s = lax.axis_index("s")
    wid = s * NC + c
    out_base = wid * HALF

    pltpu.sync_copy(desc_hbm.at[wid], descv)
    v = descv[...]
    lanes = lax.iota(jnp.int32, 16)
    zero16 = jnp.zeros((16,), jnp.int32)
    src0 = jnp.sum(jnp.where(lanes == 0, v, zero16))
    nvalid = jnp.sum(jnp.where(lanes == 1, v, zero16))

    big = nvalid >= CH
    n_chunks = jnp.where(big, (nvalid + CH - 1) // CH, 0)
    lastoff = jnp.maximum(nvalid - CH, 0)

    def off(j):
        return jnp.minimum(j * CH, lastoff)

    @pl.when(n_chunks > 0)
    def _():
        pltpu.async_copy(flat_hbm.at[pl.ds(src0 + off(0), CH)], buf_a, sem_ra)

    @pl.when(n_chunks > 1)
    def _():
        pltpu.async_copy(flat_hbm.at[pl.ds(src0 + off(1), CH)], buf_b, sem_rb)

    zrow = jnp.zeros((16,), jnp.float32)

    def zr(i, carry):
        for k in range(D // 16):
            zbuf[i, pl.ds(k * 16, 16)] = zrow
        return carry

    lax.fori_loop(0, CH, zr, 0)

    nzero = HALF - nvalid
    nz_full = nzero // CH
    zstart = out_base + nvalid

    def zf(i, carry):
        pltpu.async_copy(zbuf, out_hbm.at[pl.ds(zstart + i * CH, CH)], sem_z)
        return carry

    lax.fori_loop(0, nz_full, zf, 0)

    ztail = nzero - nz_full * CH
    zt0 = zstart + nz_full * CH
    for kb in LADDER:
        zoff = (ztail // (2 * kb)) * (2 * kb)

        @pl.when((ztail & kb) != 0)
        def _(zoff=zoff, kb=kb):
            pltpu.async_copy(zbuf.at[pl.ds(0, kb)],
                             out_hbm.at[pl.ds(zt0 + zoff, kb)], sem_z)

    def wait_read(buf, sem):
        pltpu.make_async_copy(flat_hbm.at[pl.ds(0, CH)], buf, sem).wait()

    def wait_write(buf, sem):
        pltpu.make_async_copy(buf, out_hbm.at[pl.ds(out_base, CH)],
                              sem).wait()

    def grp(g, carry):
        j0 = 2 * g
        j1 = 2 * g + 1

        @pl.when(j0 < n_chunks)
        def _():
            wait_read(buf_a, sem_ra)
            pltpu.async_copy(buf_a, out_hbm.at[pl.ds(out_base + off(j0), CH)],
                             sem_wa)

        @pl.when(j1 < n_chunks)
        def _():
            wait_read(buf_b, sem_rb)
            pltpu.async_copy(buf_b, out_hbm.at[pl.ds(out_base + off(j1), CH)],
                             sem_wb)

        @pl.when(j0 + 2 < n_chunks)
        def _():
            wait_write(buf_a, sem_wa)
            pltpu.async_copy(flat_hbm.at[pl.ds(src0 + off(j0 + 2), CH)],
                             buf_a, sem_ra)

        @pl.when(j1 + 2 < n_chunks)
        def _():
            wait_write(buf_b, sem_wb)
            pltpu.async_copy(flat_hbm.at[pl.ds(src0 + off(j1 + 2), CH)],
                             buf_b, sem_rb)

        return carry

    lax.fori_loop(0, (n_chunks + 1) // 2, grp, 0)

    for kb in LADDER:
        loff = (nvalid // (2 * kb)) * (2 * kb)

        @pl.when(jnp.logical_and(jnp.logical_not(big), (nvalid & kb) != 0))
        def _(loff=loff, kb=kb):
            pltpu.async_copy(flat_hbm.at[pl.ds(src0 + loff, kb)],
                             buf_a.at[pl.ds(loff, kb)], sem_l).wait()
            pltpu.async_copy(buf_a.at[pl.ds(loff, kb)],
                             out_hbm.at[pl.ds(out_base + loff, kb)],
                             sem_l).wait()

    @pl.when(n_chunks >= 1)
    def _():
        wait_write(buf_a, sem_wa)

    @pl.when(n_chunks >= 2)
    def _():
        wait_write(buf_b, sem_wb)

    def zd(i, carry):
        pltpu.make_async_copy(zbuf, out_hbm.at[pl.ds(out_base, CH)],
                              sem_z).wait()
        return carry

    lax.fori_loop(0, nz_full, zd, 0)

    for kb in LADDER:
        @pl.when((ztail & kb) != 0)
        def _(kb=kb):
            pltpu.make_async_copy(zbuf.at[pl.ds(0, kb)],
                                  out_hbm.at[pl.ds(out_base, kb)],
                                  sem_z).wait()


@jax.jit
def kernel(flat, cu_seqlens):
    cu = cu_seqlens.astype(jnp.int32)
    w = jnp.arange(NW, dtype=jnp.int32)
    b = w // 2
    p0 = (w % 2) * HALF
    src0 = cu[b] + p0
    nvalid = jnp.clip(cu[b + 1] - cu[b] - p0, 0, HALF)
    desc = jnp.zeros((NW, 16), jnp.int32)
    desc = desc.at[:, 0].set(src0).at[:, 1].set(nvalid)

    run = pl.kernel(
        _body,
        out_type=jax.ShapeDtypeStruct((B * MAX_LEN, D), jnp.float32),
        mesh=plsc.VectorSubcoreMesh(core_axis_name="c", subcore_axis_name="s"),
        scratch_types=[
            pltpu.VMEM((16,), jnp.int32),
            pltpu.VMEM((CH, D), jnp.float32),
            pltpu.VMEM((CH, D), jnp.float32),
            pltpu.VMEM((CH, D), jnp.float32),
            pltpu.SemaphoreType.DMA,
            pltpu.SemaphoreType.DMA,
            pltpu.SemaphoreType.DMA,
            pltpu.SemaphoreType.DMA,
            pltpu.SemaphoreType.DMA,
            pltpu.SemaphoreType.DMA,
        ],
        compiler_params=pltpu.CompilerParams(
            use_tc_tiling_on_sc=False, needs_layout_passes=False),
    )
    dense = run(flat, desc)
    return dense.reshape(B, MAX_LEN, D)
